# Initial kernel scaffold; baseline (speedup 1.0000x reference)
import jax
import jax.numpy as jnp
from jax.experimental import pallas as pl


def kernel(x, router_wq, router_wk, Wq, bq, Wk, bk, Wv, bv, Wo, bo, W1, b1, W2, b2, ln1_w, ln1_b, ln2_w, ln2_b):
    raise NotImplementedError("write your pallas kernel here")


# speedup vs baseline: 2.7789x; 2.7789x over previous
import math
from functools import partial

import jax
import jax.numpy as jnp
from jax.experimental import pallas as pl
from jax.experimental.pallas import tpu as pltpu
from jax.experimental.pallas import tpu_sc as plsc

F32 = jnp.float32
BF16 = jnp.bfloat16
_SPARSITY = 0.1
_NUM_HEADS = 16
_LN_EPS = 1e-5


def _colsum_body(x_ref, wq_ref, s_ref):
    t = pl.program_id(1)
    ql = jax.lax.dot_general(
        x_ref[0].astype(BF16), wq_ref[...].astype(BF16),
        (((1,), (1,)), ((), ())), preferred_element_type=F32)
    part = jnp.sum(ql, axis=0, keepdims=True)[None]

    @pl.when(t == 0)
    def _():
        s_ref[...] = part

    @pl.when(t != 0)
    def _():
        s_ref[...] += part


def _scores_body(x_ref, s_ref, wk_ref, o_ref, *, n_tokens):
    qg = (s_ref[0] * (1.0 / n_tokens)).astype(BF16)
    kl = jax.lax.dot_general(
        x_ref[0].astype(BF16), wk_ref[...].astype(BF16),
        (((1,), (1,)), ((), ())), preferred_element_type=F32)
    prod = kl.astype(BF16).astype(F32) * qg.astype(F32)
    s = jnp.sum(prod, axis=1, keepdims=True)
    o_ref[0] = s.T


def _ln_body(x_ref, w_ref, b_ref, o_ref):
    x = x_ref[0]
    mu = jnp.mean(x, axis=1, keepdims=True)
    xc = x - mu
    var = jnp.mean(xc * xc, axis=1, keepdims=True)
    o_ref[0] = xc / jnp.sqrt(var + _LN_EPS) * w_ref[...] + b_ref[...]


def _sc_gather(flat, gidx, kp_tot, d):
    mesh = plsc.VectorSubcoreMesh(core_axis_name="core", subcore_axis_name="subcore")
    gw = 128

    @partial(
        pl.kernel,
        out_type=jax.ShapeDtypeStruct((kp_tot, d), F32),
        mesh=mesh,
    )
    def kern(x_hbm, i_hbm, o_hbm):
        def body(i_vmem, o_vmem):
            pltpu.sync_copy(x_hbm.at[i_vmem.at[0]], o_vmem)

        pltpu.emit_pipeline(
            body,
            grid=(kp_tot // gw,),
            in_specs=[pl.BlockSpec((1, gw), index_map=lambda i: (0, i))],
            out_specs=[pl.BlockSpec((gw, d), index_map=lambda i: (i, 0))],
            core_axis_name="subcore",
            dimension_semantics=(pltpu.PARALLEL,),
        )(i_hbm, o_hbm)

    return kern(flat, gidx)


def _attn_body(xs_ref, wq_ref, wk_ref, wv_ref, wo_ref,
               bq_ref, bk_ref, bv_ref, bo_ref, d_ref, *, kk, heads):
    xs = xs_ref[0]
    kp, d = xs.shape
    hd = d // heads
    xsb = xs.astype(BF16)
    cdim = (((1,), (1,)), ((), ()))

    q = jax.lax.dot_general(xsb, wq_ref[...], cdim, preferred_element_type=F32)
    q = (q + bq_ref[...]) * (1.0 / math.sqrt(hd))
    k = jax.lax.dot_general(xsb, wk_ref[...], cdim, preferred_element_type=F32)
    k = k + bk_ref[...]
    v = jax.lax.dot_general(xsb, wv_ref[...], cdim, preferred_element_type=F32)
    v = v + bv_ref[...]

    col = jax.lax.broadcasted_iota(jnp.int32, (kp, kp), 1)
    neg = jnp.float32(-1e30)
    outs = []
    for h in range(heads):
        sl = slice(h * hd, (h + 1) * hd)
        qh = q[:, sl].astype(BF16)
        kh = k[:, sl].astype(BF16)
        s = jax.lax.dot_general(qh, kh, cdim, preferred_element_type=F32)
        s = jnp.where(col < kk, s, neg)
        m = jnp.max(s, axis=1, keepdims=True)
        p = jnp.exp(s - m)
        p = p / jnp.sum(p, axis=1, keepdims=True)
        vh = v[:, sl].astype(BF16)
        oh = jax.lax.dot_general(
            p.astype(BF16), vh, (((1,), (0,)), ((), ())),
            preferred_element_type=F32)
        outs.append(oh)
    o = jnp.concatenate(outs, axis=1)
    y = jax.lax.dot_general(o.astype(BF16), wo_ref[...], cdim,
                            preferred_element_type=F32) + bo_ref[...]
    row = jax.lax.broadcasted_iota(jnp.int32, (kp, d), 0)
    d_ref[0] = jnp.where(row < kk, y - xs, 0.0)


def _ffn_body(x_ref, nx_ref, idxf_ref, delta_ref, w_ref, b_ref,
              w1_ref, b1_ref, w2_ref, b2_ref, o_ref, *, tn, n_chunks):
    t = pl.program_id(1)
    x = x_ref[0]
    nx = nx_ref[0]
    kp = idxf_ref.shape[-1]

    rows = jax.lax.broadcasted_iota(jnp.int32, (tn, kp), 0) + t * tn
    pmat = (rows == idxf_ref[0]).astype(BF16)
    delta = delta_ref[0]
    dhi = delta.astype(BF16)
    dlo = (delta - dhi.astype(F32)).astype(BF16)
    cdim = (((1,), (0,)), ((), ()))
    scat = (jax.lax.dot_general(pmat, dhi, cdim, preferred_element_type=F32)
            + jax.lax.dot_general(pmat, dlo, cdim, preferred_element_type=F32))

    xn = x + nx + scat

    mu = jnp.mean(xn, axis=1, keepdims=True)
    xc = xn - mu
    var = jnp.mean(xc * xc, axis=1, keepdims=True)
    xb = (xc / jnp.sqrt(var + _LN_EPS) * w_ref[...] + b_ref[...]).astype(BF16)

    d = x.shape[1]
    hidden = w1_ref.shape[0]
    fb = hidden // n_chunks
    ccdim = (((1,), (1,)), ((), ()))
    acc = jnp.zeros((tn, d), F32)
    inv_sqrt2 = 1.0 / math.sqrt(2.0)
    for f in range(n_chunks):
        w1c = w1_ref[f * fb:(f + 1) * fb, :]
        h = jax.lax.dot_general(xb, w1c, ccdim, preferred_element_type=F32)
        h = h + b1_ref[:, f * fb:(f + 1) * fb]
        h = 0.5 * h * (1.0 + jax.lax.erf(h * inv_sqrt2))
        w2c = w2_ref[:, f * fb:(f + 1) * fb]
        acc += jax.lax.dot_general(h.astype(BF16), w2c, ccdim,
                                   preferred_element_type=F32)
    o_ref[0] = xn + acc + b2_ref[...]


def kernel(x, router_wq, router_wk, Wq, bq, Wk, bk, Wv, bv, Wo, bo,
           W1, b1, W2, b2, ln1_w, ln1_b, ln2_w, ln2_b):
    b, n, d = x.shape
    l = Wq.shape[0]
    kk = max(1, int(n * _SPARSITY))
    kp = ((kk + 127) // 128) * 128
    tn = min(2048, n)
    tf = min(512, n)
    n_chunks = max(1, W1.shape[1] // 1024)

    rank = router_wq.shape[0]
    rp = max(128, rank)
    wq_pad = jnp.pad(router_wq, ((0, rp - rank), (0, 0)))
    wk_pad = jnp.pad(router_wk, ((0, rp - rank), (0, 0)))
    sums = pl.pallas_call(
        _colsum_body,
        grid=(b, n // tn),
        in_specs=[
            pl.BlockSpec((1, tn, d), lambda i, t: (i, t, 0)),
            pl.BlockSpec((rp, d), lambda i, t: (0, 0)),
        ],
        out_specs=pl.BlockSpec((1, 1, rp), lambda i, t: (i, 0, 0)),
        out_shape=jax.ShapeDtypeStruct((b, 1, rp), F32),
    )(x, wq_pad)
    scores3 = pl.pallas_call(
        partial(_scores_body, n_tokens=n),
        grid=(b, n // tn),
        in_specs=[
            pl.BlockSpec((1, tn, d), lambda i, t: (i, t, 0)),
            pl.BlockSpec((1, 1, rp), lambda i, t: (i, 0, 0)),
            pl.BlockSpec((rp, d), lambda i, t: (0, 0)),
        ],
        out_specs=pl.BlockSpec((1, 1, tn), lambda i, t: (i, 0, t)),
        out_shape=jax.ShapeDtypeStruct((b, 1, n), F32),
    )(x, sums, wk_pad)
    _, idx = jax.lax.top_k(scores3[:, 0, :], kk)

    idx_pad = jnp.concatenate(
        [idx, jnp.broadcast_to(idx[:, :1], (b, kp - kk))], axis=1)
    gidx = (idx_pad + (jnp.arange(b, dtype=idx_pad.dtype) * n)[:, None])
    splits = 4
    gidx = (gidx.reshape(b * kp, 1) * splits
            + jnp.arange(splits, dtype=idx_pad.dtype)[None, :])
    gidx = gidx.reshape(1, b * kp * splits).astype(jnp.int32)
    idxf = idx_pad.astype(jnp.int32).reshape(b, 1, kp)

    wq_b = Wq.astype(BF16)
    wk_b = Wk.astype(BF16)
    wv_b = Wv.astype(BF16)
    wo_b = Wo.astype(BF16)
    w1_b = W1.astype(BF16)
    w2_b = W2.astype(BF16)
    bq2, bk2, bv2, bo2 = (z.reshape(l, 1, d) for z in (bq, bk, bv, bo))
    b12 = b1.reshape(l, 1, -1)
    b22 = b2.reshape(l, 1, d)
    ln1w2, ln1b2 = ln1_w.reshape(l, 1, d), ln1_b.reshape(l, 1, d)
    ln2w2, ln2b2 = ln2_w.reshape(l, 1, d), ln2_b.reshape(l, 1, d)

    ln_call = pl.pallas_call(
        _ln_body,
        grid=(b, n // tn),
        in_specs=[
            pl.BlockSpec((1, tn, d), lambda i, t: (i, t, 0)),
            pl.BlockSpec((1, d), lambda i, t: (0, 0)),
            pl.BlockSpec((1, d), lambda i, t: (0, 0)),
        ],
        out_specs=pl.BlockSpec((1, tn, d), lambda i, t: (i, t, 0)),
        out_shape=jax.ShapeDtypeStruct((b, n, d), F32),
    )

    attn_call = pl.pallas_call(
        partial(_attn_body, kk=kk, heads=_NUM_HEADS),
        grid=(b,),
        in_specs=[
            pl.BlockSpec((1, kp, d), lambda i: (i, 0, 0)),
            *[pl.BlockSpec((d, d), lambda i: (0, 0))] * 4,
            *[pl.BlockSpec((1, d), lambda i: (0, 0))] * 4,
        ],
        out_specs=pl.BlockSpec((1, kp, d), lambda i: (i, 0, 0)),
        out_shape=jax.ShapeDtypeStruct((b, kp, d), F32),
    )

    hidden = W1.shape[1]
    ffn_call = pl.pallas_call(
        partial(_ffn_body, tn=tf, n_chunks=n_chunks),
        grid=(b, n // tf),
        in_specs=[
            pl.BlockSpec((1, tf, d), lambda i, t: (i, t, 0)),
            pl.BlockSpec((1, tf, d), lambda i, t: (i, t, 0)),
            pl.BlockSpec((1, 1, kp), lambda i, t: (i, 0, 0)),
            pl.BlockSpec((1, kp, d), lambda i, t: (i, 0, 0)),
            pl.BlockSpec((1, d), lambda i, t: (0, 0)),
            pl.BlockSpec((1, d), lambda i, t: (0, 0)),
            pl.BlockSpec((hidden, d), lambda i, t: (0, 0)),
            pl.BlockSpec((1, hidden), lambda i, t: (0, 0)),
            pl.BlockSpec((d, hidden), lambda i, t: (0, 0)),
            pl.BlockSpec((1, d), lambda i, t: (0, 0)),
        ],
        out_specs=pl.BlockSpec((1, tf, d), lambda i, t: (i, t, 0)),
        out_shape=jax.ShapeDtypeStruct((b, n, d), F32),
    )

    for i in range(l):
        nx = ln_call(x, ln1w2[i], ln1b2[i])
        xsp = _sc_gather(nx.reshape(b * n * splits, d // splits), gidx,
                         b * kp * splits, d // splits)
        xsp = xsp.reshape(b, kp, d)
        delta = attn_call(xsp, wq_b[i], wk_b[i], wv_b[i], wo_b[i],
                          bq2[i], bk2[i], bv2[i], bo2[i])
        x = ffn_call(x, nx, idxf, delta, ln2w2[i], ln2b2[i],
                     w1_b[i], b12[i], w2_b[i], b22[i])
    return x


# speedup vs baseline: 2.9588x; 1.0647x over previous
import math
from functools import partial

import jax
import jax.numpy as jnp
from jax.experimental import pallas as pl
from jax.experimental.pallas import tpu as pltpu
from jax.experimental.pallas import tpu_sc as plsc

F32 = jnp.float32
BF16 = jnp.bfloat16
_SPARSITY = 0.1
_NUM_HEADS = 16
_LN_EPS = 1e-5


def _colsum_body(x_ref, wq_ref, s_ref):
    t = pl.program_id(1)
    ql = jax.lax.dot_general(
        x_ref[0].astype(BF16), wq_ref[...].astype(BF16),
        (((1,), (1,)), ((), ())), preferred_element_type=F32)
    part = jnp.sum(ql, axis=0, keepdims=True)[None]

    @pl.when(t == 0)
    def _():
        s_ref[...] = part

    @pl.when(t != 0)
    def _():
        s_ref[...] += part


def _scores_body(x_ref, s_ref, wk_ref, o_ref, *, n_tokens):
    qg = (s_ref[0] * (1.0 / n_tokens)).astype(BF16)
    kl = jax.lax.dot_general(
        x_ref[0].astype(BF16), wk_ref[...].astype(BF16),
        (((1,), (1,)), ((), ())), preferred_element_type=F32)
    prod = kl.astype(BF16).astype(F32) * qg.astype(F32)
    s = jnp.sum(prod, axis=1, keepdims=True)
    o_ref[0] = s.T


def _ln_body(x_ref, w_ref, b_ref, o_ref):
    x = x_ref[0]
    mu = jnp.mean(x, axis=1, keepdims=True)
    xc = x - mu
    var = jnp.mean(xc * xc, axis=1, keepdims=True)
    o_ref[0] = xc / jnp.sqrt(var + _LN_EPS) * w_ref[...] + b_ref[...]


def _sc_gather(flat, gidx, kp_tot, d):
    mesh = plsc.VectorSubcoreMesh(core_axis_name="core", subcore_axis_name="subcore")
    gw = 128

    @partial(
        pl.kernel,
        out_type=jax.ShapeDtypeStruct((kp_tot, d), F32),
        mesh=mesh,
    )
    def kern(x_hbm, i_hbm, o_hbm):
        def body(i_vmem, o_vmem):
            pltpu.sync_copy(x_hbm.at[i_vmem.at[0]], o_vmem)

        pltpu.emit_pipeline(
            body,
            grid=(kp_tot // gw,),
            in_specs=[pl.BlockSpec((1, gw), index_map=lambda i: (0, i))],
            out_specs=[pl.BlockSpec((gw, d), index_map=lambda i: (i, 0))],
            core_axis_name="subcore",
            dimension_semantics=(pltpu.PARALLEL,),
        )(i_hbm, o_hbm)

    return kern(flat, gidx)


def _attn_body(xs_ref, wq_ref, wk_ref, wv_ref, wo_ref,
               bq_ref, bk_ref, bv_ref, bo_ref, lw_ref, lb_ref, d_ref,
               *, kk, heads):
    xr = xs_ref[0]
    kp, d = xr.shape
    hd = d // heads
    mu = jnp.mean(xr, axis=1, keepdims=True)
    xc = xr - mu
    var = jnp.mean(xc * xc, axis=1, keepdims=True)
    xs = xc / jnp.sqrt(var + _LN_EPS) * lw_ref[...] + lb_ref[...]
    xsb = xs.astype(BF16)
    cdim = (((1,), (1,)), ((), ()))

    q = jax.lax.dot_general(xsb, wq_ref[...], cdim, preferred_element_type=F32)
    q = (q + bq_ref[...]) * (1.0 / math.sqrt(hd))
    k = jax.lax.dot_general(xsb, wk_ref[...], cdim, preferred_element_type=F32)
    k = k + bk_ref[...]
    v = jax.lax.dot_general(xsb, wv_ref[...], cdim, preferred_element_type=F32)
    v = v + bv_ref[...]

    col = jax.lax.broadcasted_iota(jnp.int32, (kp, kp), 1)
    neg = jnp.float32(-1e30)
    outs = []
    for h in range(heads):
        sl = slice(h * hd, (h + 1) * hd)
        qh = q[:, sl].astype(BF16)
        kh = k[:, sl].astype(BF16)
        s = jax.lax.dot_general(qh, kh, cdim, preferred_element_type=F32)
        s = jnp.where(col < kk, s, neg)
        m = jnp.max(s, axis=1, keepdims=True)
        p = jnp.exp(s - m)
        p = p / jnp.sum(p, axis=1, keepdims=True)
        vh = v[:, sl].astype(BF16)
        oh = jax.lax.dot_general(
            p.astype(BF16), vh, (((1,), (0,)), ((), ())),
            preferred_element_type=F32)
        outs.append(oh)
    o = jnp.concatenate(outs, axis=1)
    y = jax.lax.dot_general(o.astype(BF16), wo_ref[...], cdim,
                            preferred_element_type=F32) + bo_ref[...]
    row = jax.lax.broadcasted_iota(jnp.int32, (kp, d), 0)
    d_ref[0] = jnp.where(row < kk, y - xs, 0.0)


def _ffn_body(x_ref, idxf_ref, delta_ref, lw_ref, lb_ref, w_ref, b_ref,
              w1_ref, b1_ref, w2_ref, b2_ref, o_ref, *, tn, n_chunks):
    t = pl.program_id(1)
    x = x_ref[0]
    mu1 = jnp.mean(x, axis=1, keepdims=True)
    xc1 = x - mu1
    var1 = jnp.mean(xc1 * xc1, axis=1, keepdims=True)
    nx = xc1 / jnp.sqrt(var1 + _LN_EPS) * lw_ref[...] + lb_ref[...]
    kp = idxf_ref.shape[-1]

    rows = jax.lax.broadcasted_iota(jnp.int32, (tn, kp), 0) + t * tn
    pmat = (rows == idxf_ref[0]).astype(BF16)
    delta = delta_ref[0]
    dhi = delta.astype(BF16)
    dlo = (delta - dhi.astype(F32)).astype(BF16)
    cdim = (((1,), (0,)), ((), ()))
    scat = (jax.lax.dot_general(pmat, dhi, cdim, preferred_element_type=F32)
            + jax.lax.dot_general(pmat, dlo, cdim, preferred_element_type=F32))

    xn = x + nx + scat

    mu = jnp.mean(xn, axis=1, keepdims=True)
    xc = xn - mu
    var = jnp.mean(xc * xc, axis=1, keepdims=True)
    xb = (xc / jnp.sqrt(var + _LN_EPS) * w_ref[...] + b_ref[...]).astype(BF16)

    d = x.shape[1]
    hidden = w1_ref.shape[0]
    fb = hidden // n_chunks
    ccdim = (((1,), (1,)), ((), ()))
    acc = jnp.zeros((tn, d), F32)
    inv_sqrt2 = 1.0 / math.sqrt(2.0)
    for f in range(n_chunks):
        w1c = w1_ref[f * fb:(f + 1) * fb, :]
        h = jax.lax.dot_general(xb, w1c, ccdim, preferred_element_type=F32)
        h = h + b1_ref[:, f * fb:(f + 1) * fb]
        h = 0.5 * h * (1.0 + jax.lax.erf(h * inv_sqrt2))
        w2c = w2_ref[:, f * fb:(f + 1) * fb]
        acc += jax.lax.dot_general(h.astype(BF16), w2c, ccdim,
                                   preferred_element_type=F32)
    o_ref[0] = xn + acc + b2_ref[...]


def kernel(x, router_wq, router_wk, Wq, bq, Wk, bk, Wv, bv, Wo, bo,
           W1, b1, W2, b2, ln1_w, ln1_b, ln2_w, ln2_b):
    b, n, d = x.shape
    l = Wq.shape[0]
    kk = max(1, int(n * _SPARSITY))
    kp = ((kk + 127) // 128) * 128
    tn = min(2048, n)
    tf = min(512, n)
    n_chunks = max(1, W1.shape[1] // 1024)

    rank = router_wq.shape[0]
    rp = max(128, rank)
    wq_pad = jnp.pad(router_wq, ((0, rp - rank), (0, 0)))
    wk_pad = jnp.pad(router_wk, ((0, rp - rank), (0, 0)))
    sums = pl.pallas_call(
        _colsum_body,
        grid=(b, n // tn),
        in_specs=[
            pl.BlockSpec((1, tn, d), lambda i, t: (i, t, 0)),
            pl.BlockSpec((rp, d), lambda i, t: (0, 0)),
        ],
        out_specs=pl.BlockSpec((1, 1, rp), lambda i, t: (i, 0, 0)),
        out_shape=jax.ShapeDtypeStruct((b, 1, rp), F32),
    )(x, wq_pad)
    scores3 = pl.pallas_call(
        partial(_scores_body, n_tokens=n),
        grid=(b, n // tn),
        in_specs=[
            pl.BlockSpec((1, tn, d), lambda i, t: (i, t, 0)),
            pl.BlockSpec((1, 1, rp), lambda i, t: (i, 0, 0)),
            pl.BlockSpec((rp, d), lambda i, t: (0, 0)),
        ],
        out_specs=pl.BlockSpec((1, 1, tn), lambda i, t: (i, 0, t)),
        out_shape=jax.ShapeDtypeStruct((b, 1, n), F32),
    )(x, sums, wk_pad)
    _, idx = jax.lax.top_k(scores3[:, 0, :], kk)

    idx_pad = jnp.concatenate(
        [idx, jnp.broadcast_to(idx[:, :1], (b, kp - kk))], axis=1)
    gidx = (idx_pad + (jnp.arange(b, dtype=idx_pad.dtype) * n)[:, None])
    splits = 4
    gidx = (gidx.reshape(b * kp, 1) * splits
            + jnp.arange(splits, dtype=idx_pad.dtype)[None, :])
    gidx = gidx.reshape(1, b * kp * splits).astype(jnp.int32)
    idxf = idx_pad.astype(jnp.int32).reshape(b, 1, kp)

    wq_b = Wq.astype(BF16)
    wk_b = Wk.astype(BF16)
    wv_b = Wv.astype(BF16)
    wo_b = Wo.astype(BF16)
    w1_b = W1.astype(BF16)
    w2_b = W2.astype(BF16)
    bq2, bk2, bv2, bo2 = (z.reshape(l, 1, d) for z in (bq, bk, bv, bo))
    b12 = b1.reshape(l, 1, -1)
    b22 = b2.reshape(l, 1, d)
    ln1w2, ln1b2 = ln1_w.reshape(l, 1, d), ln1_b.reshape(l, 1, d)
    ln2w2, ln2b2 = ln2_w.reshape(l, 1, d), ln2_b.reshape(l, 1, d)

    attn_call = pl.pallas_call(
        partial(_attn_body, kk=kk, heads=_NUM_HEADS),
        grid=(b,),
        in_specs=[
            pl.BlockSpec((1, kp, d), lambda i: (i, 0, 0)),
            *[pl.BlockSpec((d, d), lambda i: (0, 0))] * 4,
            *[pl.BlockSpec((1, d), lambda i: (0, 0))] * 6,
        ],
        out_specs=pl.BlockSpec((1, kp, d), lambda i: (i, 0, 0)),
        out_shape=jax.ShapeDtypeStruct((b, kp, d), F32),
    )

    hidden = W1.shape[1]
    ffn_call = pl.pallas_call(
        partial(_ffn_body, tn=tf, n_chunks=n_chunks),
        grid=(b, n // tf),
        in_specs=[
            pl.BlockSpec((1, tf, d), lambda i, t: (i, t, 0)),
            pl.BlockSpec((1, 1, kp), lambda i, t: (i, 0, 0)),
            pl.BlockSpec((1, kp, d), lambda i, t: (i, 0, 0)),
            pl.BlockSpec((1, d), lambda i, t: (0, 0)),
            pl.BlockSpec((1, d), lambda i, t: (0, 0)),
            pl.BlockSpec((1, d), lambda i, t: (0, 0)),
            pl.BlockSpec((1, d), lambda i, t: (0, 0)),
            pl.BlockSpec((hidden, d), lambda i, t: (0, 0)),
            pl.BlockSpec((1, hidden), lambda i, t: (0, 0)),
            pl.BlockSpec((d, hidden), lambda i, t: (0, 0)),
            pl.BlockSpec((1, d), lambda i, t: (0, 0)),
        ],
        out_specs=pl.BlockSpec((1, tf, d), lambda i, t: (i, t, 0)),
        out_shape=jax.ShapeDtypeStruct((b, n, d), F32),
    )

    for i in range(l):
        xsp = _sc_gather(x.reshape(b * n * splits, d // splits), gidx,
                         b * kp * splits, d // splits)
        xsp = xsp.reshape(b, kp, d)
        delta = attn_call(xsp, wq_b[i], wk_b[i], wv_b[i], wo_b[i],
                          bq2[i], bk2[i], bv2[i], bo2[i],
                          ln1w2[i], ln1b2[i])
        x = ffn_call(x, idxf, delta, ln1w2[i], ln1b2[i],
                     ln2w2[i], ln2b2[i], w1_b[i], b12[i], w2_b[i], b22[i])
    return x


# speedup vs baseline: 2.9733x; 1.0049x over previous
import math
from functools import partial

import jax
import jax.numpy as jnp
from jax.experimental import pallas as pl
from jax.experimental.pallas import tpu as pltpu
from jax.experimental.pallas import tpu_sc as plsc

F32 = jnp.float32
BF16 = jnp.bfloat16
_SPARSITY = 0.1
_NUM_HEADS = 16
_LN_EPS = 1e-5


def _colsum_body(x_ref, wq_ref, s_ref):
    t = pl.program_id(1)
    ql = jax.lax.dot_general(
        x_ref[0].astype(BF16), wq_ref[...].astype(BF16),
        (((1,), (1,)), ((), ())), preferred_element_type=F32)
    part = jnp.sum(ql, axis=0, keepdims=True)[None]

    @pl.when(t == 0)
    def _():
        s_ref[...] = part

    @pl.when(t != 0)
    def _():
        s_ref[...] += part


def _scores_body(x_ref, s_ref, wk_ref, o_ref, *, n_tokens):
    qg = (s_ref[0] * (1.0 / n_tokens)).astype(BF16)
    kl = jax.lax.dot_general(
        x_ref[0].astype(BF16), wk_ref[...].astype(BF16),
        (((1,), (1,)), ((), ())), preferred_element_type=F32)
    prod = kl.astype(BF16).astype(F32) * qg.astype(F32)
    s = jnp.sum(prod, axis=1, keepdims=True)
    o_ref[0] = s.T


def _ln_body(x_ref, w_ref, b_ref, o_ref):
    x = x_ref[0]
    mu = jnp.mean(x, axis=1, keepdims=True)
    xc = x - mu
    var = jnp.mean(xc * xc, axis=1, keepdims=True)
    o_ref[0] = xc / jnp.sqrt(var + _LN_EPS) * w_ref[...] + b_ref[...]


def _sc_gather(flat, gidx, kp_tot, d):
    mesh = plsc.VectorSubcoreMesh(core_axis_name="core", subcore_axis_name="subcore")
    gw = 128

    @partial(
        pl.kernel,
        out_type=jax.ShapeDtypeStruct((kp_tot, d), F32),
        mesh=mesh,
    )
    def kern(x_hbm, i_hbm, o_hbm):
        def body(i_vmem, o_vmem):
            pltpu.sync_copy(x_hbm.at[i_vmem.at[0]], o_vmem)

        pltpu.emit_pipeline(
            body,
            grid=(kp_tot // gw,),
            in_specs=[pl.BlockSpec((1, gw), index_map=lambda i: (0, i))],
            out_specs=[pl.BlockSpec((gw, d), index_map=lambda i: (i, 0))],
            core_axis_name="subcore",
            dimension_semantics=(pltpu.PARALLEL,),
        )(i_hbm, o_hbm)

    return kern(flat, gidx)


def _attn_body(xs_ref, wq_ref, wk_ref, wv_ref, wo_ref,
               bq_ref, bk_ref, bv_ref, bo_ref, lw_ref, lb_ref, d_ref,
               *, kk, heads):
    xr = xs_ref[0]
    kp, d = xr.shape
    hd = d // heads
    mu = jnp.mean(xr, axis=1, keepdims=True)
    xc = xr - mu
    var = jnp.mean(xc * xc, axis=1, keepdims=True)
    xs = xc / jnp.sqrt(var + _LN_EPS) * lw_ref[...] + lb_ref[...]
    xsb = xs
    cdim = (((1,), (1,)), ((), ()))

    q = jax.lax.dot_general(xsb, wq_ref[...], cdim, preferred_element_type=F32)
    q = (q + bq_ref[...]) * (1.0 / math.sqrt(hd))
    k = jax.lax.dot_general(xsb, wk_ref[...], cdim, preferred_element_type=F32)
    k = k + bk_ref[...]
    v = jax.lax.dot_general(xsb, wv_ref[...], cdim, preferred_element_type=F32)
    v = v + bv_ref[...]

    col = jax.lax.broadcasted_iota(jnp.int32, (kp, kp), 1)
    neg = jnp.float32(-1e30)
    outs = []
    for h in range(heads):
        sl = slice(h * hd, (h + 1) * hd)
        qh = q[:, sl]
        kh = k[:, sl]
        s = jax.lax.dot_general(qh, kh, cdim, preferred_element_type=F32)
        s = jnp.where(col < kk, s, neg)
        m = jnp.max(s, axis=1, keepdims=True)
        p = jnp.exp(s - m)
        p = p / jnp.sum(p, axis=1, keepdims=True)
        vh = v[:, sl]
        oh = jax.lax.dot_general(
            p, vh, (((1,), (0,)), ((), ())),
            preferred_element_type=F32)
        outs.append(oh)
    o = jnp.concatenate(outs, axis=1)
    y = jax.lax.dot_general(o, wo_ref[...], cdim,
                            preferred_element_type=F32) + bo_ref[...]
    row = jax.lax.broadcasted_iota(jnp.int32, (kp, d), 0)
    d_ref[0] = jnp.where(row < kk, y - xs, 0.0)


def _ffn_body(x_ref, idxf_ref, delta_ref, lw_ref, lb_ref, w_ref, b_ref,
              w1_ref, b1_ref, w2_ref, b2_ref, o_ref, *, tn, n_chunks):
    t = pl.program_id(1)
    x = x_ref[0]
    mu1 = jnp.mean(x, axis=1, keepdims=True)
    xc1 = x - mu1
    var1 = jnp.mean(xc1 * xc1, axis=1, keepdims=True)
    nx = xc1 / jnp.sqrt(var1 + _LN_EPS) * lw_ref[...] + lb_ref[...]
    kp = idxf_ref.shape[-1]

    rows = jax.lax.broadcasted_iota(jnp.int32, (tn, kp), 0) + t * tn
    pmat = (rows == idxf_ref[0]).astype(BF16)
    delta = delta_ref[0]
    dhi = delta.astype(BF16)
    dlo = (delta - dhi.astype(F32)).astype(BF16)
    cdim = (((1,), (0,)), ((), ()))
    scat = (jax.lax.dot_general(pmat, dhi, cdim, preferred_element_type=F32)
            + jax.lax.dot_general(pmat, dlo, cdim, preferred_element_type=F32))

    xn = x + nx + scat

    mu = jnp.mean(xn, axis=1, keepdims=True)
    xc = xn - mu
    var = jnp.mean(xc * xc, axis=1, keepdims=True)
    xb = xc / jnp.sqrt(var + _LN_EPS) * w_ref[...] + b_ref[...]

    d = x.shape[1]
    hidden = w1_ref.shape[0]
    fb = hidden // n_chunks
    ccdim = (((1,), (1,)), ((), ()))
    acc = jnp.zeros((tn, d), F32)
    inv_sqrt2 = 1.0 / math.sqrt(2.0)
    for f in range(n_chunks):
        w1c = w1_ref[f * fb:(f + 1) * fb, :]
        h = jax.lax.dot_general(xb, w1c, ccdim, preferred_element_type=F32)
        h = h + b1_ref[:, f * fb:(f + 1) * fb]
        h = 0.5 * h * (1.0 + jax.lax.erf(h * inv_sqrt2))
        w2c = w2_ref[:, f * fb:(f + 1) * fb]
        acc += jax.lax.dot_general(h, w2c, ccdim,
                                   preferred_element_type=F32)
    o_ref[0] = xn + acc + b2_ref[...]


def kernel(x, router_wq, router_wk, Wq, bq, Wk, bk, Wv, bv, Wo, bo,
           W1, b1, W2, b2, ln1_w, ln1_b, ln2_w, ln2_b):
    b, n, d = x.shape
    l = Wq.shape[0]
    kk = max(1, int(n * _SPARSITY))
    kp = ((kk + 127) // 128) * 128
    tn = min(2048, n)
    tf = min(512, n)
    n_chunks = max(1, W1.shape[1] // 1024)

    rank = router_wq.shape[0]
    rp = max(128, rank)
    wq_pad = jnp.pad(router_wq, ((0, rp - rank), (0, 0)))
    wk_pad = jnp.pad(router_wk, ((0, rp - rank), (0, 0)))
    sums = pl.pallas_call(
        _colsum_body,
        grid=(b, n // tn),
        in_specs=[
            pl.BlockSpec((1, tn, d), lambda i, t: (i, t, 0)),
            pl.BlockSpec((rp, d), lambda i, t: (0, 0)),
        ],
        out_specs=pl.BlockSpec((1, 1, rp), lambda i, t: (i, 0, 0)),
        out_shape=jax.ShapeDtypeStruct((b, 1, rp), F32),
    )(x, wq_pad)
    scores3 = pl.pallas_call(
        partial(_scores_body, n_tokens=n),
        grid=(b, n // tn),
        in_specs=[
            pl.BlockSpec((1, tn, d), lambda i, t: (i, t, 0)),
            pl.BlockSpec((1, 1, rp), lambda i, t: (i, 0, 0)),
            pl.BlockSpec((rp, d), lambda i, t: (0, 0)),
        ],
        out_specs=pl.BlockSpec((1, 1, tn), lambda i, t: (i, 0, t)),
        out_shape=jax.ShapeDtypeStruct((b, 1, n), F32),
    )(x, sums, wk_pad)
    _, idx = jax.lax.top_k(scores3[:, 0, :], kk)

    idx_pad = jnp.concatenate(
        [idx, jnp.broadcast_to(idx[:, :1], (b, kp - kk))], axis=1)
    gidx = (idx_pad + (jnp.arange(b, dtype=idx_pad.dtype) * n)[:, None])
    splits = 4
    gidx = (gidx.reshape(b * kp, 1) * splits
            + jnp.arange(splits, dtype=idx_pad.dtype)[None, :])
    gidx = gidx.reshape(1, b * kp * splits).astype(jnp.int32)
    idxf = idx_pad.astype(jnp.int32).reshape(b, 1, kp)

    wq_b, wk_b, wv_b, wo_b, w1_b, w2_b = Wq, Wk, Wv, Wo, W1, W2
    bq2, bk2, bv2, bo2 = (z.reshape(l, 1, d) for z in (bq, bk, bv, bo))
    b12 = b1.reshape(l, 1, -1)
    b22 = b2.reshape(l, 1, d)
    ln1w2, ln1b2 = ln1_w.reshape(l, 1, d), ln1_b.reshape(l, 1, d)
    ln2w2, ln2b2 = ln2_w.reshape(l, 1, d), ln2_b.reshape(l, 1, d)

    attn_call = pl.pallas_call(
        partial(_attn_body, kk=kk, heads=_NUM_HEADS),
        grid=(b,),
        in_specs=[
            pl.BlockSpec((1, kp, d), lambda i: (i, 0, 0)),
            *[pl.BlockSpec((d, d), lambda i: (0, 0))] * 4,
            *[pl.BlockSpec((1, d), lambda i: (0, 0))] * 6,
        ],
        out_specs=pl.BlockSpec((1, kp, d), lambda i: (i, 0, 0)),
        out_shape=jax.ShapeDtypeStruct((b, kp, d), F32),
    )

    hidden = W1.shape[1]
    ffn_call = pl.pallas_call(
        partial(_ffn_body, tn=tf, n_chunks=n_chunks),
        grid=(b, n // tf),
        in_specs=[
            pl.BlockSpec((1, tf, d), lambda i, t: (i, t, 0)),
            pl.BlockSpec((1, 1, kp), lambda i, t: (i, 0, 0)),
            pl.BlockSpec((1, kp, d), lambda i, t: (i, 0, 0)),
            pl.BlockSpec((1, d), lambda i, t: (0, 0)),
            pl.BlockSpec((1, d), lambda i, t: (0, 0)),
            pl.BlockSpec((1, d), lambda i, t: (0, 0)),
            pl.BlockSpec((1, d), lambda i, t: (0, 0)),
            pl.BlockSpec((hidden, d), lambda i, t: (0, 0)),
            pl.BlockSpec((1, hidden), lambda i, t: (0, 0)),
            pl.BlockSpec((d, hidden), lambda i, t: (0, 0)),
            pl.BlockSpec((1, d), lambda i, t: (0, 0)),
        ],
        out_specs=pl.BlockSpec((1, tf, d), lambda i, t: (i, t, 0)),
        out_shape=jax.ShapeDtypeStruct((b, n, d), F32),
    )

    for i in range(l):
        xsp = _sc_gather(x.reshape(b * n * splits, d // splits), gidx,
                         b * kp * splits, d // splits)
        xsp = xsp.reshape(b, kp, d)
        delta = attn_call(xsp, wq_b[i], wk_b[i], wv_b[i], wo_b[i],
                          bq2[i], bk2[i], bv2[i], bo2[i],
                          ln1w2[i], ln1b2[i])
        x = ffn_call(x, idxf, delta, ln1w2[i], ln1b2[i],
                     ln2w2[i], ln2b2[i], w1_b[i], b12[i], w2_b[i], b22[i])
    return x
